# baseline (speedup 1.0000x reference)
import functools

import jax
import jax.numpy as jnp
from jax import lax
from jax.experimental import pallas as pl
from jax.experimental.pallas import tpu as pltpu
from jax.experimental.pallas import tpu_sc as plsc

BATCH = 16384
HIST = 200
D = 64
B = BATCH * HIST

_info = plsc.get_sparse_core_info()
NC, NS, NL = _info.num_cores, _info.num_subcores, _info.num_lanes
NW = NC * NS
B_PER_W = B // NW
CHUNK = 256
N_BLOCKS = B_PER_W // CHUNK
BG_PER_CHUNK = CHUNK // 128

assert B % (8 * NW) == 0
assert B_PER_W % CHUNK == 0
assert BATCH % CHUNK == 0
assert N_BLOCKS % 2 == 0 and N_BLOCKS >= 4


def _sc_lookup(x_hmajor, emb):
    mesh = plsc.VectorSubcoreMesh(core_axis_name="c", subcore_axis_name="s")

    @functools.partial(
        pl.kernel,
        mesh=mesh,
        out_type=jax.ShapeDtypeStruct((HIST, D // 8, BATCH // 128, 8, 128),
                                      jnp.float32),
        scratch_types=[
            pltpu.VMEM((CHUNK,), jnp.int32),
            pltpu.VMEM((CHUNK,), jnp.int32),
            pltpu.VMEM((CHUNK, D), jnp.float32),
            pltpu.VMEM((CHUNK, D), jnp.float32),
            pltpu.VMEM((D // 8, BG_PER_CHUNK, 8, 128), jnp.float32),
            pltpu.VMEM((D // 8, BG_PER_CHUNK, 8, 128), jnp.float32),
            pltpu.SemaphoreType.DMA,
            pltpu.SemaphoreType.DMA,
            pltpu.SemaphoreType.DMA,
            pltpu.SemaphoreType.DMA,
            pltpu.SemaphoreType.DMA,
        ],
        compiler_params=pltpu.CompilerParams(use_tc_tiling_on_sc=False,
                                             needs_layout_passes=False),
    )
    def body(x_hbm, emb_hbm, out_hbm, idx0, idx1, rows0, rows1, tr0, tr1,
             s_i0, s_i1, s_g, s_st0, s_st1):
        wid = lax.axis_index("s") * NC + lax.axis_index("c")
        base = wid * B_PER_W
        idx_v = (idx0, idx1)
        rows_v = (rows0, rows1)
        tr_v = (tr0, tr1)
        s_i = (s_i0, s_i1)
        s_st = (s_st0, s_st1)

        iota = lax.iota(jnp.int32, NL)
        rbase = [iota + (bgp * 128 + brg * 16)
                 for bgp in range(BG_PER_CHUNK) for brg in range(8)]

        def idx_start(i, s):
            pltpu.async_copy(x_hbm.at[pl.ds(base + i * CHUNK, CHUNK)],
                             idx_v[s], s_i[s])

        def idx_wait(s):
            pltpu.make_async_copy(x_hbm.at[pl.ds(base, CHUNK)],
                                  idx_v[s], s_i[s]).wait()

        def gather_start(s):
            pltpu.async_copy(emb_hbm.at[idx_v[s]], rows_v[s], s_g)

        def gather_wait(s):
            pltpu.make_async_copy(emb_hbm.at[idx_v[s]], rows_v[s],
                                  s_g).wait()

        def out_slice(i):
            j0 = base + i * CHUNK
            h = j0 // BATCH
            bg0 = (j0 % BATCH) // 128
            return out_hbm.at[h, :, pl.ds(bg0, BG_PER_CHUNK)]

        def store_start(i, s):
            pltpu.async_copy(tr_v[s], out_slice(i), s_st[s])

        def store_wait(s):
            pltpu.make_async_copy(tr_v[s], out_slice(0), s_st[s]).wait()

        def transpose(s):
            rows, tr = rows_v[s], tr_v[s]

            def dg_body(dg, carry):
                for bgp in range(BG_PER_CHUNK):
                    for dr in range(8):
                        col = jnp.full((NL,), dg * 8 + dr, jnp.int32)
                        for brg in range(8):
                            v = plsc.load_gather(
                                rows, [rbase[bgp * 8 + brg], col])
                            tr[dg, bgp, dr, pl.ds(brg * NL, NL)] = v
                return carry

            lax.fori_loop(0, D // 8, dg_body, 0, unroll=False)

        def block(i, s):
            gather_wait(s)

            @pl.when(i + 1 < N_BLOCKS)
            def _():
                idx_wait(1 - s)
                gather_start(1 - s)

            @pl.when(i + 2 < N_BLOCKS)
            def _():
                idx_start(i + 2, s)

            @pl.when(i >= 2)
            def _():
                store_wait(s)

            transpose(s)
            store_start(i, s)

        idx_start(0, 0)
        idx_start(1, 1)
        idx_wait(0)
        gather_start(0)

        def group(g, carry):
            block(2 * g, 0)
            block(2 * g + 1, 1)
            return carry

        lax.fori_loop(0, N_BLOCKS // 2, group, 0, unroll=False)

        store_wait(0)
        store_wait(1)

    return body(x_hmajor, emb)


def kernel(x, emb):
    x_flat = x.astype(jnp.int32).transpose(1, 0).reshape(B)
    out5 = _sc_lookup(x_flat, emb)
    return out5.transpose(2, 4, 0, 1, 3).reshape(BATCH, HIST, D)


# speedup vs baseline: 1.6203x; 1.6203x over previous
import functools

import jax
import jax.numpy as jnp
from jax import lax
from jax.experimental import pallas as pl
from jax.experimental.pallas import tpu as pltpu
from jax.experimental.pallas import tpu_sc as plsc

BATCH = 16384
HIST = 200
D = 64
B = BATCH * HIST

_info = plsc.get_sparse_core_info()
NC, NS, NL = _info.num_cores, _info.num_subcores, _info.num_lanes
NW = NC * NS
B_PER_W = B // NW
CHUNK = 256
N_BLOCKS = B_PER_W // CHUNK
BG_PER_CHUNK = CHUNK // 128

assert B % (8 * NW) == 0
assert B_PER_W % CHUNK == 0
assert BATCH % CHUNK == 0
assert N_BLOCKS % 2 == 0 and N_BLOCKS >= 4


def _sc_lookup(x_hmajor, emb):
    mesh = plsc.VectorSubcoreMesh(core_axis_name="c", subcore_axis_name="s")

    @functools.partial(
        pl.kernel,
        mesh=mesh,
        out_type=jax.ShapeDtypeStruct((HIST, D // 8, BATCH // 128, 8, 128),
                                      jnp.float32),
        scratch_types=[
            pltpu.VMEM((CHUNK,), jnp.int32),
            pltpu.VMEM((CHUNK,), jnp.int32),
            pltpu.VMEM((CHUNK, D), jnp.float32),
            pltpu.VMEM((CHUNK, D), jnp.float32),
            pltpu.VMEM((D // 8, BG_PER_CHUNK, 8, 128), jnp.float32),
            pltpu.VMEM((D // 8, BG_PER_CHUNK, 8, 128), jnp.float32),
            pltpu.SemaphoreType.DMA,
            pltpu.SemaphoreType.DMA,
            pltpu.SemaphoreType.DMA,
            pltpu.SemaphoreType.DMA,
            pltpu.SemaphoreType.DMA,
        ],
        compiler_params=pltpu.CompilerParams(use_tc_tiling_on_sc=False,
                                             needs_layout_passes=False),
    )
    def body(x_hbm, emb_hbm, out_hbm, idx0, idx1, rows0, rows1, tr0, tr1,
             s_i0, s_i1, s_g, s_st0, s_st1):
        wid = lax.axis_index("s") * NC + lax.axis_index("c")
        base = wid * B_PER_W
        idx_v = (idx0, idx1)
        rows_v = (rows0, rows1)
        tr_v = (tr0, tr1)
        s_i = (s_i0, s_i1)
        s_st = (s_st0, s_st1)

        iota = lax.iota(jnp.int32, NL)
        rbase = [iota + (bgp * 128 + brg * 16)
                 for bgp in range(BG_PER_CHUNK) for brg in range(8)]

        def idx_start(i, s):
            pltpu.async_copy(x_hbm.at[pl.ds(base + i * CHUNK, CHUNK)],
                             idx_v[s], s_i[s])

        def idx_wait(s):
            pltpu.make_async_copy(x_hbm.at[pl.ds(base, CHUNK)],
                                  idx_v[s], s_i[s]).wait()

        def gather_start(s):
            pltpu.async_copy(emb_hbm.at[idx_v[s]], rows_v[s], s_g)

        def gather_wait(s):
            pltpu.make_async_copy(emb_hbm.at[idx_v[s]], rows_v[s],
                                  s_g).wait()

        def out_slice(i):
            j0 = base + i * CHUNK
            h = j0 // BATCH
            bg0 = (j0 % BATCH) // 128
            return out_hbm.at[h, :, pl.ds(bg0, BG_PER_CHUNK)]

        def store_start(i, s):
            pltpu.async_copy(tr_v[s], out_slice(i), s_st[s])

        def store_wait(s):
            pltpu.make_async_copy(tr_v[s], out_slice(0), s_st[s]).wait()

        def transpose(s):
            rows, tr = rows_v[s], tr_v[s]

            @plsc.parallel_loop(0, D // 8)
            def dg_body(dg):
                for bgp in range(BG_PER_CHUNK):
                    for dr in range(8):
                        col = jnp.full((NL,), dg * 8 + dr, jnp.int32)
                        for brg in range(8):
                            v = plsc.load_gather(
                                rows, [rbase[bgp * 8 + brg], col])
                            tr[dg, bgp, dr, pl.ds(brg * NL, NL)] = v

        def block(i, s):
            gather_wait(s)

            @pl.when(i + 1 < N_BLOCKS)
            def _():
                idx_wait(1 - s)
                gather_start(1 - s)

            @pl.when(i + 2 < N_BLOCKS)
            def _():
                idx_start(i + 2, s)

            @pl.when(i >= 2)
            def _():
                store_wait(s)

            transpose(s)
            store_start(i, s)

        idx_start(0, 0)
        idx_start(1, 1)
        idx_wait(0)
        gather_start(0)

        def group(g, carry):
            block(2 * g, 0)
            block(2 * g + 1, 1)
            return carry

        lax.fori_loop(0, N_BLOCKS // 2, group, 0, unroll=False)

        store_wait(0)
        store_wait(1)

    return body(x_hmajor, emb)


def kernel(x, emb):
    x_flat = x.astype(jnp.int32).transpose(1, 0).reshape(B)
    out5 = _sc_lookup(x_flat, emb)
    return out5.transpose(2, 4, 0, 1, 3).reshape(BATCH, HIST, D)


# speedup vs baseline: 1.8429x; 1.1374x over previous
import functools

import jax
import jax.numpy as jnp
from jax import lax
from jax.experimental import pallas as pl
from jax.experimental.pallas import tpu as pltpu
from jax.experimental.pallas import tpu_sc as plsc

BATCH = 16384
HIST = 200
D = 64
B = BATCH * HIST

_info = plsc.get_sparse_core_info()
NC, NS, NL = _info.num_cores, _info.num_subcores, _info.num_lanes
NW = NC * NS
B_PER_W = B // NW
CHUNK = 256
N_BLOCKS = B_PER_W // CHUNK
BG_PER_CHUNK = CHUNK // 128

assert B % (8 * NW) == 0
assert B_PER_W % CHUNK == 0
assert BATCH % CHUNK == 0
assert N_BLOCKS % 2 == 0 and N_BLOCKS >= 4


def _sc_lookup(x_hmajor, emb):
    mesh = plsc.VectorSubcoreMesh(core_axis_name="c", subcore_axis_name="s")

    @functools.partial(
        pl.kernel,
        mesh=mesh,
        out_type=jax.ShapeDtypeStruct((HIST, D // 8, BATCH // 128, 8, 128),
                                      jnp.float32),
        scratch_types=[
            pltpu.VMEM((CHUNK,), jnp.int32),
            pltpu.VMEM((CHUNK,), jnp.int32),
            pltpu.VMEM((CHUNK, D), jnp.float32),
            pltpu.VMEM((CHUNK, D), jnp.float32),
            pltpu.VMEM((D // 8, BG_PER_CHUNK, 8, 128), jnp.float32),
            pltpu.VMEM((D // 8, BG_PER_CHUNK, 8, 128), jnp.float32),
            pltpu.SemaphoreType.DMA,
            pltpu.SemaphoreType.DMA,
            pltpu.SemaphoreType.DMA,
            pltpu.SemaphoreType.DMA,
            pltpu.SemaphoreType.DMA,
        ],
        compiler_params=pltpu.CompilerParams(use_tc_tiling_on_sc=False,
                                             needs_layout_passes=False),
    )
    def body(x_hbm, emb_hbm, out_hbm, idx0, idx1, rows0, rows1, tr0, tr1,
             s_i0, s_i1, s_g, s_st0, s_st1):
        wid = lax.axis_index("s") * NC + lax.axis_index("c")
        base = wid * B_PER_W
        idx_v = (idx0, idx1)
        rows_v = (rows0, rows1)
        tr_v = (tr0, tr1)
        s_i = (s_i0, s_i1)
        s_st = (s_st0, s_st1)

        iota = lax.iota(jnp.int32, NL)
        rbase = [iota + (bgp * 128 + brg * 16)
                 for bgp in range(BG_PER_CHUNK) for brg in range(8)]

        def idx_start(i, s):
            pltpu.async_copy(x_hbm.at[pl.ds(base + i * CHUNK, CHUNK)],
                             idx_v[s], s_i[s])

        def idx_wait(s):
            pltpu.make_async_copy(x_hbm.at[pl.ds(base, CHUNK)],
                                  idx_v[s], s_i[s]).wait()

        def gather_start(s):
            pltpu.async_copy(emb_hbm.at[idx_v[s]], rows_v[s], s_g)

        def gather_wait(s):
            pltpu.make_async_copy(emb_hbm.at[idx_v[s]], rows_v[s],
                                  s_g).wait()

        def out_slice(i):
            j0 = base + i * CHUNK
            h = j0 // BATCH
            bg0 = (j0 % BATCH) // 128
            return out_hbm.at[h, :, pl.ds(bg0, BG_PER_CHUNK)]

        def store_start(i, s):
            pltpu.async_copy(tr_v[s], out_slice(i), s_st[s])

        def store_wait(s):
            pltpu.make_async_copy(tr_v[s], out_slice(0), s_st[s]).wait()

        def transpose(s):
            rows, tr = rows_v[s], tr_v[s]

            @plsc.parallel_loop(0, D * BG_PER_CHUNK, unroll=4)
            def t_body(t):
                dg = t >> 4
                bgp = (t >> 3) & 1
                dr = t & 7
                col = jnp.full((NL,), dg * 8 + dr, jnp.int32)
                rb = jnp.full((NL,), bgp * 128, jnp.int32)
                for brg in range(8):
                    v = plsc.load_gather(rows, [rbase[brg] + rb, col])
                    tr[dg, bgp, dr, pl.ds(brg * NL, NL)] = v

        def block(i, s):
            gather_wait(s)

            @pl.when(i + 1 < N_BLOCKS)
            def _():
                idx_wait(1 - s)
                gather_start(1 - s)

            @pl.when(i + 2 < N_BLOCKS)
            def _():
                idx_start(i + 2, s)

            @pl.when(i >= 2)
            def _():
                store_wait(s)

            transpose(s)
            store_start(i, s)

        idx_start(0, 0)
        idx_start(1, 1)
        idx_wait(0)
        gather_start(0)

        def group(g, carry):
            block(2 * g, 0)
            block(2 * g + 1, 1)
            return carry

        lax.fori_loop(0, N_BLOCKS // 2, group, 0, unroll=False)

        store_wait(0)
        store_wait(1)

    return body(x_hmajor, emb)


def kernel(x, emb):
    x_flat = x.astype(jnp.int32).transpose(1, 0).reshape(B)
    out5 = _sc_lookup(x_flat, emb)
    return out5.transpose(2, 4, 0, 1, 3).reshape(BATCH, HIST, D)


# speedup vs baseline: 4.6397x; 2.5175x over previous
import functools

import jax
import jax.numpy as jnp
from jax import lax
from jax.experimental import pallas as pl
from jax.experimental.pallas import tpu as pltpu
from jax.experimental.pallas import tpu_sc as plsc

BATCH = 16384
HIST = 200
D = 64
B = BATCH * HIST

_info = plsc.get_sparse_core_info()
NC, NS, NL = _info.num_cores, _info.num_subcores, _info.num_lanes
NW = NC * NS
B_PER_W = B // NW
CHUNK = 256
N_BLOCKS = B_PER_W // CHUNK
BG_PER_CHUNK = CHUNK // 128

assert B % (8 * NW) == 0
assert B_PER_W % CHUNK == 0
assert BATCH % CHUNK == 0
assert N_BLOCKS % 2 == 0 and N_BLOCKS >= 4


def _sc_lookup(x_hmajor, emb):
    mesh = plsc.VectorSubcoreMesh(core_axis_name="c", subcore_axis_name="s")

    @functools.partial(
        pl.kernel,
        mesh=mesh,
        out_type=jax.ShapeDtypeStruct((HIST, D // 8, BATCH // 128, 8, 128),
                                      jnp.float32),
        scratch_types=[
            pltpu.VMEM((CHUNK,), jnp.int32),
            pltpu.VMEM((CHUNK,), jnp.int32),
            pltpu.VMEM((CHUNK, D), jnp.float32),
            pltpu.VMEM((CHUNK, D), jnp.float32),
            pltpu.VMEM((D * BG_PER_CHUNK, 131), jnp.float32),
            pltpu.VMEM((D * BG_PER_CHUNK, 131), jnp.float32),
            pltpu.SemaphoreType.DMA,
            pltpu.SemaphoreType.DMA,
            pltpu.SemaphoreType.DMA,
            pltpu.SemaphoreType.DMA,
            pltpu.SemaphoreType.DMA,
        ],
        compiler_params=pltpu.CompilerParams(use_tc_tiling_on_sc=False,
                                             needs_layout_passes=False),
    )
    def body(x_hbm, emb_hbm, out_hbm, idx0, idx1, rows0, rows1, tr0, tr1,
             s_i0, s_i1, s_g, s_st0, s_st1):
        wid = lax.axis_index("s") * NC + lax.axis_index("c")
        base = wid * B_PER_W
        idx_v = (idx0, idx1)
        rows_v = (rows0, rows1)
        tr_v = (tr0, tr1)
        s_i = (s_i0, s_i1)
        s_st = (s_st0, s_st1)

        iota = lax.iota(jnp.int32, NL)
        prwc = [((c0 + iota) >> 3) * (8 * BG_PER_CHUNK) + bgp * 8
                + ((c0 + iota) & 7)
                for bgp in range(BG_PER_CHUNK)
                for c0 in range(0, D, NL)]

        def idx_start(i, s):
            pltpu.async_copy(x_hbm.at[pl.ds(base + i * CHUNK, CHUNK)],
                             idx_v[s], s_i[s])

        def idx_wait(s):
            pltpu.make_async_copy(x_hbm.at[pl.ds(base, CHUNK)],
                                  idx_v[s], s_i[s]).wait()

        def gather_start(s):
            pltpu.async_copy(emb_hbm.at[idx_v[s]], rows_v[s], s_g)

        def gather_wait(s):
            pltpu.make_async_copy(emb_hbm.at[idx_v[s]], rows_v[s],
                                  s_g).wait()

        def store_start(i, s):
            j0 = base + i * CHUNK
            h = j0 // BATCH
            bg0 = (j0 % BATCH) // 128
            for dg in range(D // 8):
                for bgp in range(BG_PER_CHUNK):
                    pltpu.async_copy(
                        tr_v[s].at[pl.ds((dg * BG_PER_CHUNK + bgp) * 8, 8),
                                   pl.ds(0, 128)],
                        out_hbm.at[h, dg, bg0 + bgp], s_st[s])

        def store_wait(s):
            for _ in range(D // 8 * BG_PER_CHUNK):
                pltpu.make_async_copy(
                    tr_v[s].at[pl.ds(0, 8), pl.ds(0, 128)],
                    out_hbm.at[0, 0, 0], s_st[s]).wait()

        def transpose(s):
            rows, tr = rows_v[s], tr_v[s]

            @plsc.parallel_loop(0, 128, unroll=2)
            def rr_body(rr):
                col = jnp.full((NL,), rr, jnp.int32)
                for bgp in range(BG_PER_CHUNK):
                    for c0g in range(D // NL):
                        v = rows[bgp * 128 + rr, pl.ds(c0g * NL, NL)]
                        plsc.store_scatter(
                            tr, [prwc[bgp * (D // NL) + c0g], col], v)

        def block(i, s):
            gather_wait(s)

            @pl.when(i + 1 < N_BLOCKS)
            def _():
                idx_wait(1 - s)
                gather_start(1 - s)

            @pl.when(i + 2 < N_BLOCKS)
            def _():
                idx_start(i + 2, s)

            @pl.when(i >= 2)
            def _():
                store_wait(s)

            transpose(s)
            store_start(i, s)

        idx_start(0, 0)
        idx_start(1, 1)
        idx_wait(0)
        gather_start(0)

        def group(g, carry):
            block(2 * g, 0)
            block(2 * g + 1, 1)
            return carry

        lax.fori_loop(0, N_BLOCKS // 2, group, 0, unroll=False)

        store_wait(0)
        store_wait(1)

    return body(x_hmajor, emb)


def kernel(x, emb):
    x_flat = x.astype(jnp.int32).transpose(1, 0).reshape(B)
    out5 = _sc_lookup(x_flat, emb)
    return out5.transpose(2, 4, 0, 1, 3).reshape(BATCH, HIST, D)


# speedup vs baseline: 4.6705x; 1.0067x over previous
import functools

import jax
import jax.numpy as jnp
from jax import lax
from jax.experimental import pallas as pl
from jax.experimental.pallas import tpu as pltpu
from jax.experimental.pallas import tpu_sc as plsc

BATCH = 16384
HIST = 200
D = 64
B = BATCH * HIST

_info = plsc.get_sparse_core_info()
NC, NS, NL = _info.num_cores, _info.num_subcores, _info.num_lanes
NW = NC * NS
B_PER_W = B // NW
CHUNK = 256
N_BLOCKS = B_PER_W // CHUNK
BG_PER_CHUNK = CHUNK // 128

assert B % (8 * NW) == 0
assert B_PER_W % CHUNK == 0
assert BATCH % CHUNK == 0
assert N_BLOCKS % 2 == 0 and N_BLOCKS >= 4


def _sc_lookup(x_hmajor, emb):
    mesh = plsc.VectorSubcoreMesh(core_axis_name="c", subcore_axis_name="s")

    @functools.partial(
        pl.kernel,
        mesh=mesh,
        out_type=jax.ShapeDtypeStruct((HIST, D // 8, BATCH // 128, 8, 128),
                                      jnp.float32),
        scratch_types=[
            pltpu.VMEM((CHUNK,), jnp.int32),
            pltpu.VMEM((CHUNK,), jnp.int32),
            pltpu.VMEM((CHUNK, D), jnp.float32),
            pltpu.VMEM((CHUNK, D), jnp.float32),
            pltpu.VMEM((D * BG_PER_CHUNK, 131), jnp.float32),
            pltpu.VMEM((D * BG_PER_CHUNK, 131), jnp.float32),
            pltpu.SemaphoreType.DMA,
            pltpu.SemaphoreType.DMA,
            pltpu.SemaphoreType.DMA,
            pltpu.SemaphoreType.DMA,
            pltpu.SemaphoreType.DMA,
        ],
        compiler_params=pltpu.CompilerParams(use_tc_tiling_on_sc=False,
                                             needs_layout_passes=False),
    )
    def body(x_hbm, emb_hbm, out_hbm, idx0, idx1, rows0, rows1, tr0, tr1,
             s_i0, s_i1, s_g, s_st0, s_st1):
        wid = lax.axis_index("s") * NC + lax.axis_index("c")
        base = wid * B_PER_W
        idx_v = (idx0, idx1)
        rows_v = (rows0, rows1)
        tr_v = (tr0, tr1)
        s_i = (s_i0, s_i1)
        s_st = (s_st0, s_st1)

        iota = lax.iota(jnp.int32, NL)
        prwc = [((c0 + iota) >> 3) * (8 * BG_PER_CHUNK) + bgp * 8
                + ((c0 + iota) & 7)
                for bgp in range(BG_PER_CHUNK)
                for c0 in range(0, D, NL)]

        def idx_start(i, s):
            j0 = base + i * CHUNK
            h = j0 // BATCH
            bg0 = (j0 % BATCH) // 128
            for q in range(BG_PER_CHUNK):
                pltpu.async_copy(x_hbm.at[h // 8, bg0 + q, h % 8],
                                 idx_v[s].at[pl.ds(q * 128, 128)], s_i[s])

        def idx_wait(s):
            for q in range(BG_PER_CHUNK):
                pltpu.make_async_copy(x_hbm.at[0, 0, 0],
                                      idx_v[s].at[pl.ds(0, 128)],
                                      s_i[s]).wait()

        def gather_start(s):
            pltpu.async_copy(emb_hbm.at[idx_v[s]], rows_v[s], s_g)

        def gather_wait(s):
            pltpu.make_async_copy(emb_hbm.at[idx_v[s]], rows_v[s],
                                  s_g).wait()

        def store_start(i, s):
            j0 = base + i * CHUNK
            h = j0 // BATCH
            bg0 = (j0 % BATCH) // 128
            for dg in range(D // 8):
                for bgp in range(BG_PER_CHUNK):
                    pltpu.async_copy(
                        tr_v[s].at[pl.ds((dg * BG_PER_CHUNK + bgp) * 8, 8),
                                   pl.ds(0, 128)],
                        out_hbm.at[h, dg, bg0 + bgp], s_st[s])

        def store_wait(s):
            for _ in range(D // 8 * BG_PER_CHUNK):
                pltpu.make_async_copy(
                    tr_v[s].at[pl.ds(0, 8), pl.ds(0, 128)],
                    out_hbm.at[0, 0, 0], s_st[s]).wait()

        def transpose(s):
            rows, tr = rows_v[s], tr_v[s]

            @plsc.parallel_loop(0, 128, unroll=2)
            def rr_body(rr):
                col = jnp.full((NL,), rr, jnp.int32)
                for bgp in range(BG_PER_CHUNK):
                    for c0g in range(D // NL):
                        v = rows[bgp * 128 + rr, pl.ds(c0g * NL, NL)]
                        plsc.store_scatter(
                            tr, [prwc[bgp * (D // NL) + c0g], col], v)

        def block(i, s):
            gather_wait(s)

            @pl.when(i + 1 < N_BLOCKS)
            def _():
                idx_wait(1 - s)
                gather_start(1 - s)

            @pl.when(i + 2 < N_BLOCKS)
            def _():
                idx_start(i + 2, s)

            @pl.when(i >= 2)
            def _():
                store_wait(s)

            transpose(s)
            store_start(i, s)

        idx_start(0, 0)
        idx_start(1, 1)
        idx_wait(0)
        gather_start(0)

        def group(g, carry):
            block(2 * g, 0)
            block(2 * g + 1, 1)
            return carry

        lax.fori_loop(0, N_BLOCKS // 2, group, 0, unroll=False)

        store_wait(0)
        store_wait(1)

    return body(x_hmajor, emb)


def kernel(x, emb):
    x4 = (x.astype(jnp.int32)
          .reshape(BATCH // 128, 128, HIST // 8, 8)
          .transpose(2, 0, 3, 1))
    out5 = _sc_lookup(x4, emb)
    return out5.transpose(2, 4, 0, 1, 3).reshape(BATCH, HIST, D)
